# baseline (speedup 1.0000x reference)
import functools

import jax
import jax.numpy as jnp
from jax import lax
from jax.experimental import pallas as pl
from jax.experimental.pallas import tpu as pltpu
from jax.experimental.pallas import tpu_sc as plsc

N = 10000
E = 160000
NC = 2
NS = 16
NW = NC * NS
NPW = 320
NPAD = NW * NPW
CH = 128
SUPC = 8
SUP = CH * SUPC
E_PAD = E + 2048
D = 128
NB = D // 16
_NEG = float("-inf")


def _make_sc_agg():
    mesh = plsc.VectorSubcoreMesh(
        core_axis_name="c", subcore_axis_name="s", num_cores=NC, num_subcores=NS
    )

    @functools.partial(
        pl.kernel,
        out_type=jax.ShapeDtypeStruct((NPAD, D), jnp.float32),
        mesh=mesh,
        scratch_types=[
            pltpu.VMEM((16,), jnp.int32),
            pltpu.VMEM((SUP,), jnp.int32),
            pltpu.VMEM((SUP,), jnp.int32),
            pltpu.VMEM((CH, D), jnp.float32),
            pltpu.VMEM((CH, D), jnp.float32),
            pltpu.VMEM((NPW, D), jnp.float32),
            pltpu.VMEM((D,), jnp.float32),
            pltpu.SemaphoreType.DMA,
            pltpu.SemaphoreType.DMA,
        ],
    )
    def sc_agg(h_hbm, srcs_hbm, dsts_hbm, starts_hbm, out_hbm,
               starts_v, src_s, dst_s, msg0, msg1, agg_v, acc_v, sem0, sem1):
        wid = lax.axis_index("c") * NS + lax.axis_index("s")
        base = wid * NPW
        pltpu.sync_copy(starts_hbm.at[pl.ds(wid * 16, 16)], starts_v)
        sv = starts_v[...]
        lo = sv[0]
        hi = sv[1]
        lo8 = (lo // 8) * 8
        n_chunks = (hi - lo8 + CH - 1) // CH
        n_super = (n_chunks + SUPC - 1) // SUPC

        neg = jnp.full((16,), _NEG, jnp.float32)

        def init_row(r, _):
            for b in range(NB):
                agg_v[r, pl.ds(b * 16, 16)] = neg
            return 0

        lax.fori_loop(0, NPW, init_row, 0)

        msgs = (msg0, msg1)
        sems = (sem0, sem1)

        GE = 32

        def compute_chunk(j, cur):
            msg = msgs[j % 2]

            def group_body(g, cur):
                accs = [acc_v[pl.ds(b * 16, 16)] for b in range(NB)]
                for half in range(GE // 16):
                    e0 = g * GE + half * 16
                    dvec = dst_s[pl.ds(j * CH + e0, 16)] - base
                    for k in range(16):
                        dl = dvec[k]
                        ok = (dl >= 0) & (dl < NPW)
                        change = ok & (dl != cur)

                        @pl.when(change & (cur >= 0))
                        def _flush(cur=cur, accs=tuple(accs)):
                            for b in range(NB):
                                agg_v[cur, pl.ds(b * 16, 16)] = accs[b]

                        for b in range(NB):
                            v = msg[e0 + k, pl.ds(b * 16, 16)]
                            v = jnp.where(ok, v, neg)
                            accs[b] = jnp.where(change, v,
                                                jnp.maximum(accs[b], v))
                        cur = jnp.where(change, dl, cur)
                for b in range(NB):
                    acc_v[pl.ds(b * 16, 16)] = accs[b]
                return cur

            return lax.fori_loop(0, CH // GE, group_body, cur)

        def super_body(s, carry):
            soff = lo8 + s * SUP
            pltpu.sync_copy(srcs_hbm.at[pl.ds(soff, SUP)], src_s)
            pltpu.sync_copy(dsts_hbm.at[pl.ds(soff, SUP)], dst_s)
            c0 = s * SUPC

            @pl.when(c0 < n_chunks)
            def _g0():
                pltpu.async_copy(
                    h_hbm.at[src_s.at[pl.ds(0, CH)]], msgs[0], sems[0])

            for j in range(SUPC):
                @pl.when(c0 + j < n_chunks)
                def _wait(j=j):
                    pltpu.make_async_copy(
                        h_hbm.at[src_s.at[pl.ds(j * CH, CH)]],
                        msgs[j % 2], sems[j % 2]).wait()

                if j + 1 < SUPC:
                    @pl.when(c0 + j + 1 < n_chunks)
                    def _issue(j=j):
                        pltpu.async_copy(
                            h_hbm.at[src_s.at[pl.ds((j + 1) * CH, CH)]],
                            msgs[(j + 1) % 2], sems[(j + 1) % 2])

                carry = compute_chunk(j, carry)
            return carry

        cur = lax.fori_loop(0, n_super, super_body, jnp.int32(-1))

        @pl.when(cur >= 0)
        def _final_flush():
            for b in range(NB):
                agg_v[cur, pl.ds(b * 16, 16)] = jnp.maximum(
                    acc_v[pl.ds(b * 16, 16)], neg)

        pltpu.sync_copy(agg_v, out_hbm.at[pl.ds(base, NPW)])

    return sc_agg


_SC_AGG = _make_sc_agg()


def _tc_layer(h, agg, wlT, bl2, wrT, last: bool):
    n, din = h.shape
    dh = wlT.shape[1]
    B = 400
    grid = (n // B,)

    def body(agg_ref, h_ref, wl_ref, wr_ref, bl_ref, o_ref):
        a = agg_ref[...]
        a = jnp.where(a == _NEG, 0.0, a)
        z = (
            jnp.dot(a, wl_ref[...], preferred_element_type=jnp.float32,
                    precision=lax.Precision.HIGHEST)
            + jnp.dot(h_ref[...], wr_ref[...], preferred_element_type=jnp.float32,
                      precision=lax.Precision.HIGHEST)
            + bl_ref[...]
        )
        if last:
            m = jnp.max(z, axis=1, keepdims=True)
            zs = z - m
            z = zs - jnp.log(jnp.sum(jnp.exp(zs), axis=1, keepdims=True))
        else:
            z = jnp.maximum(z, 0.0)
        o_ref[...] = z

    return pl.pallas_call(
        body,
        grid=grid,
        in_specs=[
            pl.BlockSpec((B, din), lambda i: (i, 0)),
            pl.BlockSpec((B, din), lambda i: (i, 0)),
            pl.BlockSpec((din, dh), lambda i: (0, 0)),
            pl.BlockSpec((din, dh), lambda i: (0, 0)),
            pl.BlockSpec((1, dh), lambda i: (0, 0)),
        ],
        out_specs=pl.BlockSpec((B, dh), lambda i: (i, 0)),
        out_shape=jax.ShapeDtypeStruct((n, dh), jnp.float32),
    )(agg, h, wlT, wrT, bl2)


def kernel(x, edge_index, Wl0, bl0, Wr0, Wl1, bl1, Wr1, Wl2, bl2, Wr2,
           Wl3, bl3, Wr3, Wl4, bl4, Wr4):
    src = edge_index[0]
    dst = edge_index[1]
    key = (dst.astype(jnp.uint32) << 18) | jnp.arange(E, dtype=jnp.uint32)
    key_sorted = lax.sort(key)
    dst_sorted = (key_sorted >> 18).astype(jnp.int32)
    src_sorted = src[(key_sorted & 0x3FFFF).astype(jnp.int32)]
    pad = E_PAD - E
    src_p = jnp.concatenate([src_sorted, jnp.zeros((pad,), jnp.int32)])
    dst_p = jnp.concatenate([dst_sorted, jnp.full((pad,), 2**30, jnp.int32)])
    bounds = (jnp.arange(33, dtype=jnp.int32) * NPW).astype(jnp.int32)
    starts = jnp.searchsorted(dst_sorted, bounds).astype(jnp.int32)
    starts_tab = jnp.zeros((NW, 16), jnp.int32)
    starts_tab = starts_tab.at[:, 0].set(starts[:NW])
    starts_tab = starts_tab.at[:, 1].set(starts[1:NW + 1])
    starts_tab = starts_tab.reshape(NW * 16)

    params = [(Wl0, bl0, Wr0), (Wl1, bl1, Wr1), (Wl2, bl2, Wr2),
              (Wl3, bl3, Wr3), (Wl4, bl4, Wr4)]
    h = x
    for i, (Wl, bl, Wr) in enumerate(params):
        d = h.shape[1]
        if d == D:
            agg = _SC_AGG(h, src_p, dst_p, starts_tab)[:N]
        else:
            halves = [
                _SC_AGG(h[:, c:c + D], src_p, dst_p, starts_tab)[:N]
                for c in range(0, d, D)
            ]
            agg = jnp.concatenate(halves, axis=1)
        h = _tc_layer(h, agg, Wl.T, bl.reshape(1, -1), Wr.T,
                      last=(i == len(params) - 1))
    return h


# speedup vs baseline: 1.0281x; 1.0281x over previous
import functools

import jax
import jax.numpy as jnp
from jax import lax
from jax.experimental import pallas as pl
from jax.experimental.pallas import tpu as pltpu
from jax.experimental.pallas import tpu_sc as plsc

N = 10000
E = 160000
NC = 2
NS = 16
NW = NC * NS
NPW = 320
NPAD = NW * NPW
CH = 128
SUPC = 8
SUP = CH * SUPC
E_PAD = E + 2048
D = 128
NB = D // 16
_NEG = float("-inf")


def _make_sc_agg():
    mesh = plsc.VectorSubcoreMesh(
        core_axis_name="c", subcore_axis_name="s", num_cores=NC, num_subcores=NS
    )

    @functools.partial(
        pl.kernel,
        out_type=jax.ShapeDtypeStruct((NPAD, D), jnp.float32),
        mesh=mesh,
        scratch_types=[
            pltpu.VMEM((16,), jnp.int32),
            pltpu.VMEM((SUP,), jnp.int32),
            pltpu.VMEM((SUP,), jnp.int32),
            pltpu.VMEM((CH, D), jnp.float32),
            pltpu.VMEM((CH, D), jnp.float32),
            pltpu.VMEM((NPW, D), jnp.float32),
            pltpu.SemaphoreType.DMA,
            pltpu.SemaphoreType.DMA,
        ],
    )
    def sc_agg(h_hbm, srcs_hbm, dsts_hbm, starts_hbm, out_hbm,
               starts_v, src_s, dst_s, msg0, msg1, agg_v, sem0, sem1):
        wid = lax.axis_index("c") * NS + lax.axis_index("s")
        base = wid * NPW
        pltpu.sync_copy(starts_hbm.at[pl.ds(wid * 16, 16)], starts_v)
        sv = starts_v[...]
        lo = sv[0]
        hi = sv[1]
        lo8 = (lo // 8) * 8
        n_chunks = (hi - lo8 + CH - 1) // CH
        n_super = (n_chunks + SUPC - 1) // SUPC

        neg = jnp.full((16,), _NEG, jnp.float32)

        def init_row(r, _):
            for b in range(NB):
                agg_v[r, pl.ds(b * 16, 16)] = neg
            return 0

        lax.fori_loop(0, NPW, init_row, 0)

        msgs = (msg0, msg1)
        sems = (sem0, sem1)

        GE = 32

        def compute_chunk(j, cur):
            msg = msgs[j % 2]

            def group_body(g, cur):
                curc = jnp.maximum(cur, 0)
                accs = [agg_v[curc, pl.ds(b * 16, 16)] for b in range(NB)]
                for half in range(GE // 16):
                    e0 = g * GE + half * 16
                    dvec = dst_s[pl.ds(j * CH + e0, 16)] - base
                    for k in range(16):
                        dl = dvec[k]
                        ok = (dl >= 0) & (dl < NPW)
                        change = ok & (dl != cur)
                        for b in range(NB):
                            v = msg[e0 + k, pl.ds(b * 16, 16)]
                            v = jnp.where(ok, v, neg)
                            accs[b] = jnp.where(change, v,
                                                jnp.maximum(accs[b], v))
                        cur = jnp.where(change, dl, cur)
                        curw = jnp.maximum(cur, 0)
                        for b in range(NB):
                            agg_v[curw, pl.ds(b * 16, 16)] = accs[b]
                return cur

            return lax.fori_loop(0, CH // GE, group_body, cur)

        def super_body(s, carry):
            soff = lo8 + s * SUP
            pltpu.sync_copy(srcs_hbm.at[pl.ds(soff, SUP)], src_s)
            pltpu.sync_copy(dsts_hbm.at[pl.ds(soff, SUP)], dst_s)
            c0 = s * SUPC

            @pl.when(c0 < n_chunks)
            def _g0():
                pltpu.async_copy(
                    h_hbm.at[src_s.at[pl.ds(0, CH)]], msgs[0], sems[0])

            for j in range(SUPC):
                @pl.when(c0 + j < n_chunks)
                def _wait(j=j):
                    pltpu.make_async_copy(
                        h_hbm.at[src_s.at[pl.ds(j * CH, CH)]],
                        msgs[j % 2], sems[j % 2]).wait()

                if j + 1 < SUPC:
                    @pl.when(c0 + j + 1 < n_chunks)
                    def _issue(j=j):
                        pltpu.async_copy(
                            h_hbm.at[src_s.at[pl.ds((j + 1) * CH, CH)]],
                            msgs[(j + 1) % 2], sems[(j + 1) % 2])

                carry = compute_chunk(j, carry)
            return carry

        lax.fori_loop(0, n_super, super_body, jnp.int32(-1))
        pltpu.sync_copy(agg_v, out_hbm.at[pl.ds(base, NPW)])

    return sc_agg


_SC_AGG = _make_sc_agg()


def _tc_layer(h, agg, wlT, bl2, wrT, last: bool):
    n, din = h.shape
    dh = wlT.shape[1]
    B = 400
    grid = (n // B,)

    def body(agg_ref, h_ref, wl_ref, wr_ref, bl_ref, o_ref):
        a = agg_ref[...]
        a = jnp.where(a == _NEG, 0.0, a)
        z = (
            jnp.dot(a, wl_ref[...], preferred_element_type=jnp.float32,
                    precision=lax.Precision.HIGHEST)
            + jnp.dot(h_ref[...], wr_ref[...], preferred_element_type=jnp.float32,
                      precision=lax.Precision.HIGHEST)
            + bl_ref[...]
        )
        if last:
            m = jnp.max(z, axis=1, keepdims=True)
            zs = z - m
            z = zs - jnp.log(jnp.sum(jnp.exp(zs), axis=1, keepdims=True))
        else:
            z = jnp.maximum(z, 0.0)
        o_ref[...] = z

    return pl.pallas_call(
        body,
        grid=grid,
        in_specs=[
            pl.BlockSpec((B, din), lambda i: (i, 0)),
            pl.BlockSpec((B, din), lambda i: (i, 0)),
            pl.BlockSpec((din, dh), lambda i: (0, 0)),
            pl.BlockSpec((din, dh), lambda i: (0, 0)),
            pl.BlockSpec((1, dh), lambda i: (0, 0)),
        ],
        out_specs=pl.BlockSpec((B, dh), lambda i: (i, 0)),
        out_shape=jax.ShapeDtypeStruct((n, dh), jnp.float32),
    )(agg, h, wlT, wrT, bl2)


def kernel(x, edge_index, Wl0, bl0, Wr0, Wl1, bl1, Wr1, Wl2, bl2, Wr2,
           Wl3, bl3, Wr3, Wl4, bl4, Wr4):
    src = edge_index[0]
    dst = edge_index[1]
    dst_sorted, src_sorted = lax.sort((dst, src), num_keys=1)
    pad = E_PAD - E
    src_p = jnp.concatenate([src_sorted, jnp.zeros((pad,), jnp.int32)])
    dst_p = jnp.concatenate([dst_sorted, jnp.full((pad,), 2**30, jnp.int32)])
    bounds = (jnp.arange(33, dtype=jnp.int32) * NPW).astype(jnp.int32)
    starts = jnp.searchsorted(dst_sorted, bounds).astype(jnp.int32)
    starts_tab = jnp.zeros((NW, 16), jnp.int32)
    starts_tab = starts_tab.at[:, 0].set(starts[:NW])
    starts_tab = starts_tab.at[:, 1].set(starts[1:NW + 1])
    starts_tab = starts_tab.reshape(NW * 16)

    params = [(Wl0, bl0, Wr0), (Wl1, bl1, Wr1), (Wl2, bl2, Wr2),
              (Wl3, bl3, Wr3), (Wl4, bl4, Wr4)]
    h = x
    for i, (Wl, bl, Wr) in enumerate(params):
        d = h.shape[1]
        if d == D:
            agg = _SC_AGG(h, src_p, dst_p, starts_tab)[:N]
        else:
            halves = [
                _SC_AGG(h[:, c:c + D], src_p, dst_p, starts_tab)[:N]
                for c in range(0, d, D)
            ]
            agg = jnp.concatenate(halves, axis=1)
        h = _tc_layer(h, agg, Wl.T, bl.reshape(1, -1), Wr.T,
                      last=(i == len(params) - 1))
    return h


# speedup vs baseline: 1.0460x; 1.0175x over previous
import functools

import jax
import jax.numpy as jnp
from jax import lax
from jax.experimental import pallas as pl
from jax.experimental.pallas import tpu as pltpu
from jax.experimental.pallas import tpu_sc as plsc

N = 10000
E = 160000
NC = 2
NS = 16
NW = NC * NS
NPW = 320
NPAD = NW * NPW
CH = 128
SUPC = 8
SUP = CH * SUPC
E_PAD = E + 4352
D = 128
NB = D // 16
_NEG = float("-inf")


def _make_sc_agg():
    mesh = plsc.VectorSubcoreMesh(
        core_axis_name="c", subcore_axis_name="s", num_cores=NC, num_subcores=NS
    )

    @functools.partial(
        pl.kernel,
        out_type=jax.ShapeDtypeStruct((NPAD, D), jnp.float32),
        mesh=mesh,
        scratch_types=[
            pltpu.VMEM((16,), jnp.int32),
            pltpu.VMEM((SUP,), jnp.int32),
            pltpu.VMEM((SUP,), jnp.int32),
            pltpu.VMEM((CH, D), jnp.float32),
            pltpu.VMEM((CH, D), jnp.float32),
            pltpu.VMEM((NPW, D), jnp.float32),
            pltpu.SemaphoreType.DMA,
            pltpu.SemaphoreType.DMA,
            pltpu.SemaphoreType.DMA,
            pltpu.SemaphoreType.DMA,
        ],
    )
    def sc_agg(h_hbm, srcs_hbm, dsts_hbm, starts_hbm, out_hbm,
               starts_v, src_s, dst_s, msg0, msg1, agg_v, sem0, sem1,
               sem2, sem3):
        wid = lax.axis_index("c") * NS + lax.axis_index("s")
        base = wid * NPW
        pltpu.sync_copy(starts_hbm.at[pl.ds(wid * 16, 16)], starts_v)
        sv = starts_v[...]
        lo = sv[0]
        hi = sv[1]
        lo8 = (lo // 8) * 8
        n_chunks = (hi - lo8 + CH - 1) // CH
        n_super = (n_chunks + SUPC - 1) // SUPC

        neg = jnp.full((16,), _NEG, jnp.float32)

        def init_row(r, _):
            for b in range(NB):
                agg_v[r, pl.ds(b * 16, 16)] = neg
            return 0

        lax.fori_loop(0, NPW, init_row, 0)

        msgs = (msg0, msg1)
        sems = (sem0, sem1)

        GE = 32

        def compute_chunk(j, cur):
            msg = msgs[j % 2]

            def group_body(g, cur):
                curc = jnp.maximum(cur, 0)
                accs = [agg_v[curc, pl.ds(b * 16, 16)] for b in range(NB)]
                for half in range(GE // 16):
                    e0 = g * GE + half * 16
                    dvec = dst_s[pl.ds(j * CH + e0, 16)] - base
                    for k in range(16):
                        dl = dvec[k]
                        ok = (dl >= 0) & (dl < NPW)
                        change = ok & (dl != cur)
                        for b in range(NB):
                            v = msg[e0 + k, pl.ds(b * 16, 16)]
                            v = jnp.where(ok, v, neg)
                            accs[b] = jnp.where(change, v,
                                                jnp.maximum(accs[b], v))
                        cur = jnp.where(change, dl, cur)
                        curw = jnp.maximum(cur, 0)
                        for b in range(NB):
                            agg_v[curw, pl.ds(b * 16, 16)] = accs[b]
                return cur

            return lax.fori_loop(0, CH // GE, group_body, cur)

        def super_body(s, carry):
            soff = lo8 + s * SUP
            pltpu.async_copy(srcs_hbm.at[pl.ds(soff, SUP)], src_s, sem2)
            pltpu.async_copy(dsts_hbm.at[pl.ds(soff, SUP)], dst_s, sem3)
            pltpu.make_async_copy(
                srcs_hbm.at[pl.ds(soff, SUP)], src_s, sem2).wait()
            pltpu.make_async_copy(
                dsts_hbm.at[pl.ds(soff, SUP)], dst_s, sem3).wait()
            c0 = s * SUPC

            @pl.when(c0 < n_chunks)
            def _g0():
                pltpu.async_copy(
                    h_hbm.at[src_s.at[pl.ds(0, CH)]], msgs[0], sems[0])

            for j in range(SUPC):
                @pl.when(c0 + j < n_chunks)
                def _wait(j=j):
                    pltpu.make_async_copy(
                        h_hbm.at[src_s.at[pl.ds(j * CH, CH)]],
                        msgs[j % 2], sems[j % 2]).wait()

                if j + 1 < SUPC:
                    @pl.when(c0 + j + 1 < n_chunks)
                    def _issue(j=j):
                        pltpu.async_copy(
                            h_hbm.at[src_s.at[pl.ds((j + 1) * CH, CH)]],
                            msgs[(j + 1) % 2], sems[(j + 1) % 2])

                carry = compute_chunk(j, carry)
            return carry

        lax.fori_loop(0, n_super, super_body, jnp.int32(-1))
        pltpu.sync_copy(agg_v, out_hbm.at[pl.ds(base, NPW)])

    return sc_agg


_SC_AGG = _make_sc_agg()


def _tc_layer(h, agg, wlT, bl2, wrT, last: bool):
    n, din = h.shape
    dh = wlT.shape[1]
    B = 400
    grid = (n // B,)

    def body(agg_ref, h_ref, wl_ref, wr_ref, bl_ref, o_ref):
        a = agg_ref[...]
        a = jnp.where(a == _NEG, 0.0, a)
        z = (
            jnp.dot(a, wl_ref[...], preferred_element_type=jnp.float32,
                    precision=lax.Precision.HIGHEST)
            + jnp.dot(h_ref[...], wr_ref[...], preferred_element_type=jnp.float32,
                      precision=lax.Precision.HIGHEST)
            + bl_ref[...]
        )
        if last:
            m = jnp.max(z, axis=1, keepdims=True)
            zs = z - m
            z = zs - jnp.log(jnp.sum(jnp.exp(zs), axis=1, keepdims=True))
        else:
            z = jnp.maximum(z, 0.0)
        o_ref[...] = z

    return pl.pallas_call(
        body,
        grid=grid,
        in_specs=[
            pl.BlockSpec((B, din), lambda i: (i, 0)),
            pl.BlockSpec((B, din), lambda i: (i, 0)),
            pl.BlockSpec((din, dh), lambda i: (0, 0)),
            pl.BlockSpec((din, dh), lambda i: (0, 0)),
            pl.BlockSpec((1, dh), lambda i: (0, 0)),
        ],
        out_specs=pl.BlockSpec((B, dh), lambda i: (i, 0)),
        out_shape=jax.ShapeDtypeStruct((n, dh), jnp.float32),
    )(agg, h, wlT, wrT, bl2)


def kernel(x, edge_index, Wl0, bl0, Wr0, Wl1, bl1, Wr1, Wl2, bl2, Wr2,
           Wl3, bl3, Wr3, Wl4, bl4, Wr4):
    src = edge_index[0]
    dst = edge_index[1]
    dst_sorted, src_sorted = lax.sort((dst, src), num_keys=1)
    pad = E_PAD - E
    src_p = jnp.concatenate([src_sorted, jnp.zeros((pad,), jnp.int32)])
    dst_p = jnp.concatenate([dst_sorted, jnp.full((pad,), 2**30, jnp.int32)])
    bounds = (jnp.arange(33, dtype=jnp.int32) * NPW).astype(jnp.int32)
    starts = jnp.searchsorted(dst_sorted, bounds).astype(jnp.int32)
    starts_tab = jnp.zeros((NW, 16), jnp.int32)
    starts_tab = starts_tab.at[:, 0].set(starts[:NW])
    starts_tab = starts_tab.at[:, 1].set(starts[1:NW + 1])
    starts_tab = starts_tab.reshape(NW * 16)

    params = [(Wl0, bl0, Wr0), (Wl1, bl1, Wr1), (Wl2, bl2, Wr2),
              (Wl3, bl3, Wr3), (Wl4, bl4, Wr4)]
    h = x
    for i, (Wl, bl, Wr) in enumerate(params):
        d = h.shape[1]
        if d == D:
            agg = _SC_AGG(h, src_p, dst_p, starts_tab)[:N]
        else:
            halves = [
                _SC_AGG(h[:, c:c + D], src_p, dst_p, starts_tab)[:N]
                for c in range(0, d, D)
            ]
            agg = jnp.concatenate(halves, axis=1)
        h = _tc_layer(h, agg, Wl.T, bl.reshape(1, -1), Wr.T,
                      last=(i == len(params) - 1))
    return h


# speedup vs baseline: 1.0585x; 1.0120x over previous
import functools

import jax
import jax.numpy as jnp
from jax import lax
from jax.experimental import pallas as pl
from jax.experimental.pallas import tpu as pltpu
from jax.experimental.pallas import tpu_sc as plsc

N = 10000
E = 160000
NC = 2
NS = 16
NW = NC * NS
NPW = 320
NPAD = NW * NPW
CH = 128
SUPC = 8
SUP = CH * SUPC
E_PAD = E + 4352
D = 128
NB = D // 16
_NEG = float("-inf")


def _make_sc_agg():
    mesh = plsc.VectorSubcoreMesh(
        core_axis_name="c", subcore_axis_name="s", num_cores=NC, num_subcores=NS
    )

    @functools.partial(
        pl.kernel,
        out_type=jax.ShapeDtypeStruct((NPAD, D), jnp.float32),
        mesh=mesh,
        scratch_types=[
            pltpu.VMEM((16,), jnp.int32),
            pltpu.VMEM((SUP,), jnp.int32),
            pltpu.VMEM((SUP,), jnp.int32),
            pltpu.VMEM((CH, D), jnp.float32),
            pltpu.VMEM((CH, D), jnp.float32),
            pltpu.VMEM((NPW, D), jnp.float32),
            pltpu.SemaphoreType.DMA,
            pltpu.SemaphoreType.DMA,
            pltpu.SemaphoreType.DMA,
            pltpu.SemaphoreType.DMA,
        ],
    )
    def sc_agg(h_hbm, srcs_hbm, dsts_hbm, starts_hbm, out_hbm,
               starts_v, src_s, dst_s, msg0, msg1, agg_v, sem0, sem1,
               sem2, sem3):
        wid = lax.axis_index("c") * NS + lax.axis_index("s")
        base = wid * NPW
        pltpu.sync_copy(starts_hbm.at[pl.ds(wid * 16, 16)], starts_v)
        sv = starts_v[...]
        lo = sv[0]
        hi = sv[1]
        lo8 = (lo // 8) * 8
        n_chunks = (hi - lo8 + CH - 1) // CH
        n_super = (n_chunks + SUPC - 1) // SUPC

        neg = jnp.full((16,), _NEG, jnp.float32)

        def init_row(r, _):
            for b in range(NB):
                agg_v[r, pl.ds(b * 16, 16)] = neg
            return 0

        lax.fori_loop(0, NPW, init_row, 0)

        msgs = (msg0, msg1)
        sems = (sem0, sem1)

        GE = 32

        def compute_chunk(j, cur):
            msg = msgs[j % 2]

            def group_body(g, cur):
                curc = jnp.maximum(cur, 0)
                accs = [agg_v[curc, pl.ds(b * 16, 16)] for b in range(NB)]
                for half in range(GE // 16):
                    e0 = g * GE + half * 16
                    dvec = dst_s[pl.ds(j * CH + e0, 16)] - base
                    for k in range(16):
                        dl = dvec[k]
                        ok = (dl >= 0) & (dl < NPW)
                        change = ok & (dl != cur)
                        for b in range(NB):
                            v = msg[e0 + k, pl.ds(b * 16, 16)]
                            v = jnp.where(ok, v, neg)
                            accs[b] = jnp.where(change, v,
                                                jnp.maximum(accs[b], v))
                        cur = jnp.where(change, dl, cur)
                        curw = jnp.maximum(cur, 0)
                        for b in range(NB):
                            agg_v[curw, pl.ds(b * 16, 16)] = accs[b]
                return cur

            return lax.fori_loop(0, CH // GE, group_body, cur)

        def super_body(s, carry):
            soff = lo8 + s * SUP
            pltpu.async_copy(srcs_hbm.at[pl.ds(soff, SUP)], src_s, sem2)
            pltpu.async_copy(dsts_hbm.at[pl.ds(soff, SUP)], dst_s, sem3)
            pltpu.make_async_copy(
                srcs_hbm.at[pl.ds(soff, SUP)], src_s, sem2).wait()
            pltpu.make_async_copy(
                dsts_hbm.at[pl.ds(soff, SUP)], dst_s, sem3).wait()
            c0 = s * SUPC

            @pl.when(c0 < n_chunks)
            def _g0():
                pltpu.async_copy(
                    h_hbm.at[src_s.at[pl.ds(0, CH)]], msgs[0], sems[0])

            for j in range(SUPC):
                @pl.when(c0 + j < n_chunks)
                def _wait(j=j):
                    pltpu.make_async_copy(
                        h_hbm.at[src_s.at[pl.ds(j * CH, CH)]],
                        msgs[j % 2], sems[j % 2]).wait()

                if j + 1 < SUPC:
                    @pl.when(c0 + j + 1 < n_chunks)
                    def _issue(j=j):
                        pltpu.async_copy(
                            h_hbm.at[src_s.at[pl.ds((j + 1) * CH, CH)]],
                            msgs[(j + 1) % 2], sems[(j + 1) % 2])

                carry = compute_chunk(j, carry)
            return carry

        lax.fori_loop(0, n_super, super_body, jnp.int32(-1))
        pltpu.sync_copy(agg_v, out_hbm.at[pl.ds(base, NPW)])

    return sc_agg


_SC_AGG = _make_sc_agg()


def _tc_r(h, wrT, bl2):
    n, din = h.shape
    dh = wrT.shape[1]
    B = 400

    def body(h_ref, wr_ref, bl_ref, o_ref):
        o_ref[...] = jnp.dot(
            h_ref[...], wr_ref[...], preferred_element_type=jnp.float32,
            precision=lax.Precision.HIGHEST) + bl_ref[...]

    return pl.pallas_call(
        body,
        grid=(n // B,),
        in_specs=[
            pl.BlockSpec((B, din), lambda i: (i, 0)),
            pl.BlockSpec((din, dh), lambda i: (0, 0)),
            pl.BlockSpec((1, dh), lambda i: (0, 0)),
        ],
        out_specs=pl.BlockSpec((B, dh), lambda i: (i, 0)),
        out_shape=jax.ShapeDtypeStruct((n, dh), jnp.float32),
    )(h, wrT, bl2)


def _tc_l(agg, r, wlT, last: bool):
    n, din = agg.shape
    dh = wlT.shape[1]
    B = 400

    def body(agg_ref, r_ref, wl_ref, o_ref):
        a = agg_ref[...]
        a = jnp.where(a == _NEG, 0.0, a)
        z = jnp.dot(a, wl_ref[...], preferred_element_type=jnp.float32,
                    precision=lax.Precision.HIGHEST) + r_ref[...]
        if last:
            m = jnp.max(z, axis=1, keepdims=True)
            zs = z - m
            z = zs - jnp.log(jnp.sum(jnp.exp(zs), axis=1, keepdims=True))
        else:
            z = jnp.maximum(z, 0.0)
        o_ref[...] = z

    return pl.pallas_call(
        body,
        grid=(n // B,),
        in_specs=[
            pl.BlockSpec((B, din), lambda i: (i, 0)),
            pl.BlockSpec((B, dh), lambda i: (i, 0)),
            pl.BlockSpec((din, dh), lambda i: (0, 0)),
        ],
        out_specs=pl.BlockSpec((B, dh), lambda i: (i, 0)),
        out_shape=jax.ShapeDtypeStruct((n, dh), jnp.float32),
    )(agg, r, wlT)


def kernel(x, edge_index, Wl0, bl0, Wr0, Wl1, bl1, Wr1, Wl2, bl2, Wr2,
           Wl3, bl3, Wr3, Wl4, bl4, Wr4):
    src = edge_index[0]
    dst = edge_index[1]
    dst_sorted, src_sorted = lax.sort((dst, src), num_keys=1)
    pad = E_PAD - E
    src_p = jnp.concatenate([src_sorted, jnp.zeros((pad,), jnp.int32)])
    dst_p = jnp.concatenate([dst_sorted, jnp.full((pad,), 2**30, jnp.int32)])
    bounds = (jnp.arange(33, dtype=jnp.int32) * NPW).astype(jnp.int32)
    starts = jnp.searchsorted(dst_sorted, bounds).astype(jnp.int32)
    starts_tab = jnp.zeros((NW, 16), jnp.int32)
    starts_tab = starts_tab.at[:, 0].set(starts[:NW])
    starts_tab = starts_tab.at[:, 1].set(starts[1:NW + 1])
    starts_tab = starts_tab.reshape(NW * 16)

    params = [(Wl0, bl0, Wr0), (Wl1, bl1, Wr1), (Wl2, bl2, Wr2),
              (Wl3, bl3, Wr3), (Wl4, bl4, Wr4)]
    h = x
    for i, (Wl, bl, Wr) in enumerate(params):
        d = h.shape[1]
        r = _tc_r(h, Wr.T, bl.reshape(1, -1))
        if d == D:
            agg = _SC_AGG(h, src_p, dst_p, starts_tab)[:N]
        else:
            halves = [
                _SC_AGG(h[:, c:c + D], src_p, dst_p, starts_tab)[:N]
                for c in range(0, d, D)
            ]
            agg = jnp.concatenate(halves, axis=1)
        h = _tc_l(agg, r, Wl.T, last=(i == len(params) - 1))
    return h
